# baseline (speedup 1.0000x reference)
import functools

import jax
import jax.numpy as jnp
from jax.experimental import pallas as pl
from jax.experimental.pallas import tpu as pltpu
from jax.experimental.pallas import tpu_sc as plsc

BATCH = 16384
NUM_CODES = 8192
CODE_DIM = 256
COMMIT_COST = 0.25

BM = 256
GATHER_W = 128


def _argmin_body(z_ref, cbt_ref, idx_ref, acc_ref):
    i = pl.program_id(0)
    z = z_ref[...]
    cbt = cbt_ref[...]
    prod = jax.lax.dot_general(
        z, cbt,
        dimension_numbers=(((1,), (0,)), ((), ())),
        preferred_element_type=jnp.float32,
        precision=jax.lax.Precision.DEFAULT,
    )
    c2 = jnp.sum(cbt * cbt, axis=0)
    z2 = jnp.sum(z * z, axis=1)
    d = (z2[:, None] - 2.0 * prod) + c2[None, :]
    m = jnp.min(d, axis=1)
    iota = jax.lax.broadcasted_iota(jnp.int32, (BM, NUM_CODES), 1)
    idx = jnp.min(jnp.where(d == m[:, None], iota, NUM_CODES), axis=1)
    idx_ref[0, 0, :] = idx

    part = jnp.sum(m).reshape(1, 1)

    @pl.when(i == 0)
    def _():
        acc_ref[...] = jnp.zeros((1, 1), jnp.float32)

    acc_ref[...] += part


def _tc_argmin(z_e, cbt):
    nb = BATCH // BM
    idx3, acc = pl.pallas_call(
        _argmin_body,
        grid=(nb,),
        in_specs=[
            pl.BlockSpec((BM, CODE_DIM), lambda i: (i, 0)),
            pl.BlockSpec((CODE_DIM, NUM_CODES), lambda i: (0, 0)),
        ],
        out_specs=[
            pl.BlockSpec((1, 1, BM), lambda i: (i, 0, 0)),
            pl.BlockSpec((1, 1), lambda i: (0, 0)),
        ],
        out_shape=[
            jax.ShapeDtypeStruct((nb, 1, BM), jnp.int32),
            jax.ShapeDtypeStruct((1, 1), jnp.float32),
        ],
    )(z_e, cbt)
    return idx3.reshape(BATCH), acc[0, 0]


def _sc_gather(codebook, indices):
    mesh = plsc.VectorSubcoreMesh(core_axis_name="core",
                                  subcore_axis_name="subcore")

    @functools.partial(
        pl.kernel,
        out_type=jax.ShapeDtypeStruct((BATCH, CODE_DIM), codebook.dtype),
        mesh=mesh,
    )
    def gather_kernel(cb_hbm, i_hbm, o_hbm):
        def body(i_vmem, o_vmem):
            pltpu.sync_copy(cb_hbm.at[i_vmem.at[0]], o_vmem)

        pltpu.emit_pipeline(
            body,
            grid=(BATCH // GATHER_W,),
            in_specs=[pl.BlockSpec((1, GATHER_W), index_map=lambda i: (0, i))],
            out_specs=[pl.BlockSpec((GATHER_W, CODE_DIM),
                                    index_map=lambda i: (i, 0))],
            core_axis_name=("core", "subcore"),
            dimension_semantics=(pltpu.PARALLEL,),
        )(i_hbm, o_hbm)

    return gather_kernel(codebook, indices.reshape(1, BATCH))


def kernel(z_e, codebook):
    cbt = codebook.T
    indices, loss_num = _tc_argmin(z_e, cbt)
    z_q = _sc_gather(codebook, indices)
    loss = (COMMIT_COST / (BATCH * CODE_DIM)) * loss_num
    return (z_q, indices, loss)


# speedup vs baseline: 1.2472x; 1.2472x over previous
import functools

import jax
import jax.numpy as jnp
from jax.experimental import pallas as pl
from jax.experimental.pallas import tpu as pltpu
from jax.experimental.pallas import tpu_sc as plsc

BATCH = 16384
NUM_CODES = 8192
CODE_DIM = 256
COMMIT_COST = 0.25

BM = 256
GATHER_W = 128


LANES = 128
NCHUNK = NUM_CODES // LANES


def _argmin_body(z_ref, cbt_ref, idx_ref, acc_ref, c2_ref):
    i = pl.program_id(0)

    @pl.when(i == 0)
    def _():
        cbt = cbt_ref[...]
        c2_ref[...] = jnp.sum(cbt * cbt, axis=0)

    z = z_ref[...]
    prod = jax.lax.dot_general(
        z, cbt_ref[...],
        dimension_numbers=(((1,), (0,)), ((), ())),
        preferred_element_type=jnp.float32,
        precision=jax.lax.Precision.DEFAULT,
    )
    z2 = jnp.sum(z * z, axis=1)

    run_m = jnp.full((BM, LANES), jnp.inf, jnp.float32)
    run_k = jnp.zeros((BM, LANES), jnp.int32)
    for k in range(NCHUNK):
        pk = prod[:, k * LANES:(k + 1) * LANES]
        ck = c2_ref[pl.ds(k * LANES, LANES)]
        dk = (z2[:, None] - 2.0 * pk) + ck[None, :]
        hit = dk < run_m
        run_k = jnp.where(hit, k, run_k)
        run_m = jnp.minimum(dk, run_m)

    m = jnp.min(run_m, axis=1)
    lane = jax.lax.broadcasted_iota(jnp.int32, (BM, LANES), 1)
    cand = run_k * LANES + lane
    idx = jnp.min(jnp.where(run_m == m[:, None], cand, NUM_CODES), axis=1)
    idx_ref[0, 0, :] = idx

    part = jnp.sum(m).reshape(1, 1)

    @pl.when(i == 0)
    def _():
        acc_ref[...] = jnp.zeros((1, 1), jnp.float32)

    acc_ref[...] += part


def _tc_argmin(z_e, cbt):
    nb = BATCH // BM
    idx3, acc = pl.pallas_call(
        _argmin_body,
        grid=(nb,),
        in_specs=[
            pl.BlockSpec((BM, CODE_DIM), lambda i: (i, 0)),
            pl.BlockSpec((CODE_DIM, NUM_CODES), lambda i: (0, 0)),
        ],
        out_specs=[
            pl.BlockSpec((1, 1, BM), lambda i: (i, 0, 0)),
            pl.BlockSpec((1, 1), lambda i: (0, 0)),
        ],
        out_shape=[
            jax.ShapeDtypeStruct((nb, 1, BM), jnp.int32),
            jax.ShapeDtypeStruct((1, 1), jnp.float32),
        ],
        scratch_shapes=[pltpu.VMEM((NUM_CODES,), jnp.float32)],
    )(z_e, cbt)
    return idx3.reshape(BATCH), acc[0, 0]


def _sc_gather(codebook, indices):
    mesh = plsc.VectorSubcoreMesh(core_axis_name="core",
                                  subcore_axis_name="subcore")

    @functools.partial(
        pl.kernel,
        out_type=jax.ShapeDtypeStruct((BATCH, CODE_DIM), codebook.dtype),
        mesh=mesh,
    )
    def gather_kernel(cb_hbm, i_hbm, o_hbm):
        def body(i_vmem, o_vmem):
            pltpu.sync_copy(cb_hbm.at[i_vmem.at[0]], o_vmem)

        pltpu.emit_pipeline(
            body,
            grid=(BATCH // GATHER_W,),
            in_specs=[pl.BlockSpec((1, GATHER_W), index_map=lambda i: (0, i))],
            out_specs=[pl.BlockSpec((GATHER_W, CODE_DIM),
                                    index_map=lambda i: (i, 0))],
            core_axis_name=("core", "subcore"),
            dimension_semantics=(pltpu.PARALLEL,),
        )(i_hbm, o_hbm)

    return gather_kernel(codebook, indices.reshape(1, BATCH))


def kernel(z_e, codebook):
    cbt = codebook.T
    indices, loss_num = _tc_argmin(z_e, cbt)
    z_q = _sc_gather(codebook, indices)
    loss = (COMMIT_COST / (BATCH * CODE_DIM)) * loss_num
    return (z_q, indices, loss)


# speedup vs baseline: 1.3352x; 1.0706x over previous
import functools

import jax
import jax.numpy as jnp
from jax.experimental import pallas as pl
from jax.experimental.pallas import tpu as pltpu
from jax.experimental.pallas import tpu_sc as plsc

BATCH = 16384
NUM_CODES = 8192
CODE_DIM = 256
COMMIT_COST = 0.25

BM = 512
GATHER_W = 128


LANES = 128
NCHUNK = NUM_CODES // LANES


def _argmin_body(z_ref, cbt_ref, idx_ref, acc_ref, c2_ref):
    i = pl.program_id(0)

    @pl.when(i == 0)
    def _():
        cbt = cbt_ref[...]
        c2_ref[...] = jnp.sum(cbt * cbt, axis=0)

    z = z_ref[...]
    prod = jax.lax.dot_general(
        z, cbt_ref[...],
        dimension_numbers=(((1,), (0,)), ((), ())),
        preferred_element_type=jnp.float32,
        precision=jax.lax.Precision.DEFAULT,
    )
    z2 = jnp.sum(z * z, axis=1)

    run_m = jnp.full((BM, LANES), jnp.inf, jnp.float32)
    run_k = jnp.zeros((BM, LANES), jnp.int32)
    for k in range(NCHUNK):
        pk = prod[:, k * LANES:(k + 1) * LANES]
        ck = c2_ref[pl.ds(k * LANES, LANES)]
        dk = (z2[:, None] - 2.0 * pk) + ck[None, :]
        hit = dk < run_m
        run_k = jnp.where(hit, k, run_k)
        run_m = jnp.minimum(dk, run_m)

    m = jnp.min(run_m, axis=1)
    lane = jax.lax.broadcasted_iota(jnp.int32, (BM, LANES), 1)
    cand = run_k * LANES + lane
    idx = jnp.min(jnp.where(run_m == m[:, None], cand, NUM_CODES), axis=1)
    idx_ref[0, 0, :] = idx

    part = jnp.sum(m).reshape(1, 1)

    @pl.when(i == 0)
    def _():
        acc_ref[...] = jnp.zeros((1, 1), jnp.float32)

    acc_ref[...] += part


def _tc_argmin(z_e, cbt):
    nb = BATCH // BM
    idx3, acc = pl.pallas_call(
        _argmin_body,
        grid=(nb,),
        in_specs=[
            pl.BlockSpec((BM, CODE_DIM), lambda i: (i, 0)),
            pl.BlockSpec((CODE_DIM, NUM_CODES), lambda i: (0, 0)),
        ],
        out_specs=[
            pl.BlockSpec((1, 1, BM), lambda i: (i, 0, 0)),
            pl.BlockSpec((1, 1), lambda i: (0, 0)),
        ],
        out_shape=[
            jax.ShapeDtypeStruct((nb, 1, BM), jnp.int32),
            jax.ShapeDtypeStruct((1, 1), jnp.float32),
        ],
        scratch_shapes=[pltpu.VMEM((NUM_CODES,), jnp.float32)],
    )(z_e, cbt)
    return idx3.reshape(BATCH), acc[0, 0]


def _sc_gather(codebook, indices):
    mesh = plsc.VectorSubcoreMesh(core_axis_name="core",
                                  subcore_axis_name="subcore")

    @functools.partial(
        pl.kernel,
        out_type=jax.ShapeDtypeStruct((BATCH, CODE_DIM), codebook.dtype),
        mesh=mesh,
    )
    def gather_kernel(cb_hbm, i_hbm, o_hbm):
        def body(i_vmem, o_vmem):
            pltpu.sync_copy(cb_hbm.at[i_vmem.at[0]], o_vmem)

        pltpu.emit_pipeline(
            body,
            grid=(BATCH // GATHER_W,),
            in_specs=[pl.BlockSpec((1, GATHER_W), index_map=lambda i: (0, i))],
            out_specs=[pl.BlockSpec((GATHER_W, CODE_DIM),
                                    index_map=lambda i: (i, 0))],
            core_axis_name=("core", "subcore"),
            dimension_semantics=(pltpu.PARALLEL,),
        )(i_hbm, o_hbm)

    return gather_kernel(codebook, indices.reshape(1, BATCH))


def kernel(z_e, codebook):
    cbt = codebook.T
    indices, loss_num = _tc_argmin(z_e, cbt)
    z_q = _sc_gather(codebook, indices)
    loss = (COMMIT_COST / (BATCH * CODE_DIM)) * loss_num
    return (z_q, indices, loss)


# speedup vs baseline: 1.4893x; 1.1154x over previous
import functools

import jax
import jax.numpy as jnp
from jax.experimental import pallas as pl
from jax.experimental.pallas import tpu as pltpu
from jax.experimental.pallas import tpu_sc as plsc

BATCH = 16384
NUM_CODES = 8192
CODE_DIM = 256
COMMIT_COST = 0.25

BM = 512
GATHER_W = 128


LANES = 128
NCHUNK = NUM_CODES // LANES


def _argmin_body(z_ref, cbt_ref, idx_ref, acc_ref, c2_ref):
    i = pl.program_id(0)

    @pl.when(i == 0)
    def _():
        cbt = cbt_ref[...]
        c2_ref[...] = jnp.sum(cbt * cbt, axis=0)

    z = z_ref[...]
    prod_m2 = jax.lax.dot_general(
        -2.0 * z, cbt_ref[...],
        dimension_numbers=(((1,), (0,)), ((), ())),
        preferred_element_type=jnp.float32,
        precision=jax.lax.Precision.DEFAULT,
    )
    z2 = jnp.sum(z * z, axis=1)

    run_m = jnp.full((BM, LANES), jnp.inf, jnp.float32)
    run_k = jnp.zeros((BM, LANES), jnp.int32)
    for k in range(NCHUNK):
        pk = prod_m2[:, k * LANES:(k + 1) * LANES]
        ck = c2_ref[pl.ds(k * LANES, LANES)]
        dk = (z2[:, None] + pk) + ck[None, :]
        hit = dk < run_m
        run_k = jnp.where(hit, k, run_k)
        run_m = jnp.minimum(dk, run_m)

    m = jnp.min(run_m, axis=1)
    lane = jax.lax.broadcasted_iota(jnp.int32, (BM, LANES), 1)
    cand = run_k * LANES + lane
    idx = jnp.min(jnp.where(run_m == m[:, None], cand, NUM_CODES), axis=1)
    idx_ref[0, 0, :] = idx

    part = jnp.sum(m).reshape(1, 1)

    @pl.when(i == 0)
    def _():
        acc_ref[...] = jnp.zeros((1, 1), jnp.float32)

    acc_ref[...] += part


def _tc_argmin(z_e, cbt):
    nb = BATCH // BM
    idx3, acc = pl.pallas_call(
        _argmin_body,
        grid=(nb,),
        in_specs=[
            pl.BlockSpec((BM, CODE_DIM), lambda i: (i, 0)),
            pl.BlockSpec((CODE_DIM, NUM_CODES), lambda i: (0, 0)),
        ],
        out_specs=[
            pl.BlockSpec((1, 1, BM), lambda i: (i, 0, 0)),
            pl.BlockSpec((1, 1), lambda i: (0, 0)),
        ],
        out_shape=[
            jax.ShapeDtypeStruct((nb, 1, BM), jnp.int32),
            jax.ShapeDtypeStruct((1, 1), jnp.float32),
        ],
        scratch_shapes=[pltpu.VMEM((NUM_CODES,), jnp.float32)],
    )(z_e, cbt)
    return idx3.reshape(BATCH), acc[0, 0]


def _sc_gather(codebook, indices):
    mesh = plsc.VectorSubcoreMesh(core_axis_name="core",
                                  subcore_axis_name="subcore")

    @functools.partial(
        pl.kernel,
        out_type=jax.ShapeDtypeStruct((BATCH, CODE_DIM), codebook.dtype),
        mesh=mesh,
    )
    def gather_kernel(cb_hbm, i_hbm, o_hbm):
        def body(i_vmem, o_vmem):
            pltpu.sync_copy(cb_hbm.at[i_vmem.at[0]], o_vmem)

        pltpu.emit_pipeline(
            body,
            grid=(BATCH // GATHER_W,),
            in_specs=[pl.BlockSpec((1, GATHER_W), index_map=lambda i: (0, i))],
            out_specs=[pl.BlockSpec((GATHER_W, CODE_DIM),
                                    index_map=lambda i: (i, 0))],
            core_axis_name=("core", "subcore"),
            dimension_semantics=(pltpu.PARALLEL,),
        )(i_hbm, o_hbm)

    return gather_kernel(codebook, indices.reshape(1, BATCH))


def kernel(z_e, codebook):
    cbt = codebook.T
    indices, loss_num = _tc_argmin(z_e, cbt)
    z_q = _sc_gather(codebook, indices)
    loss = (COMMIT_COST / (BATCH * CODE_DIM)) * loss_num
    return (z_q, indices, loss)
